# baseline (speedup 1.0000x reference)
import jax
import jax.numpy as jnp
from jax.experimental import pallas as pl
from jax.experimental.pallas import tpu as pltpu

E = 8
TOP_K = 2
TILE = 256
FF_CHUNK = 2048


def _ffn_body(te_ref, tl_ref, x_ref, W1_ref, b1_ref, W2_ref, b2_ref, o_ref):
    i = pl.program_id(0)
    ff = pl.program_id(1)

    @pl.when(tl_ref[i] != 0)
    def _compute():
        h = jnp.dot(x_ref[...].astype(jnp.bfloat16),
                    W1_ref[0].astype(jnp.bfloat16),
                    preferred_element_type=jnp.float32)
        h = h + b1_ref[0, 0][None, :]
        h = 0.5 * h * (1.0 + jax.lax.erf(h * 0.7071067811865476))
        y = jnp.dot(h.astype(jnp.bfloat16), W2_ref[0].astype(jnp.bfloat16),
                    preferred_element_type=jnp.float32)

        @pl.when(ff == 0)
        def _init():
            o_ref[...] = y + b2_ref[0, 0][None, :]

        @pl.when(ff != 0)
        def _acc():
            o_ref[...] += y


def _grouped_ffn(x_sorted, W1, b1, W2, b2, tile_expert, tile_live, n_pad):
    d_model = x_sorted.shape[1]
    d_ff = W1.shape[2]
    nt = n_pad // TILE
    nff = d_ff // FF_CHUNK
    grid_spec = pltpu.PrefetchScalarGridSpec(
        num_scalar_prefetch=2,
        grid=(nt, nff),
        in_specs=[
            pl.BlockSpec((TILE, d_model), lambda i, ff, te, tl: (i, 0)),
            pl.BlockSpec((1, d_model, FF_CHUNK),
                         lambda i, ff, te, tl: (te[i], 0, ff)),
            pl.BlockSpec((1, 1, FF_CHUNK), lambda i, ff, te, tl: (te[i], 0, ff)),
            pl.BlockSpec((1, FF_CHUNK, d_model),
                         lambda i, ff, te, tl: (te[i], ff, 0)),
            pl.BlockSpec((1, 1, d_model), lambda i, ff, te, tl: (te[i], 0, 0)),
        ],
        out_specs=pl.BlockSpec((TILE, d_model), lambda i, ff, te, tl: (i, 0)),
    )
    return pl.pallas_call(
        _ffn_body,
        grid_spec=grid_spec,
        out_shape=jax.ShapeDtypeStruct((n_pad, d_model), jnp.float32),
        compiler_params=pltpu.CompilerParams(
            dimension_semantics=("arbitrary", "arbitrary")),
    )(tile_expert, tile_live, x_sorted, W1,
      b1.reshape(b1.shape[0], 1, d_ff), W2,
      b2.reshape(b2.shape[0], 1, d_model))


def kernel(x, Wg, W1, b1, W2, b2):
    bsz, seq, d = x.shape
    x_flat = x.reshape(-1, d)
    n_tok = bsz * seq
    n_pairs = n_tok * TOP_K
    nt = n_pairs // TILE + E
    n_pad = nt * TILE

    gate_scores = x_flat @ Wg
    gate_probs = jax.nn.softmax(gate_scores, axis=-1)
    top_k_probs, top_k_idx = jax.lax.top_k(gate_probs, TOP_K)
    top_k_probs = top_k_probs / jnp.sum(top_k_probs, axis=-1, keepdims=True)

    e_flat = top_k_idx.reshape(-1).astype(jnp.int32)
    p_flat = top_k_probs.reshape(-1)
    onehot = (e_flat[:, None] == jnp.arange(E, dtype=jnp.int32)[None, :])
    onehot = onehot.astype(jnp.int32)
    rank = (jnp.cumsum(onehot, axis=0) - onehot)
    rank = (rank * onehot).sum(axis=1)
    counts = onehot.sum(axis=0)
    padded = ((counts + TILE - 1) // TILE) * TILE
    starts = jnp.concatenate(
        [jnp.zeros((1,), jnp.int32), jnp.cumsum(padded).astype(jnp.int32)])[:E]
    pos = starts[e_flat] + rank

    start_tile = starts // TILE
    tile_ids = jnp.arange(nt, dtype=jnp.int32)
    tile_expert = (tile_ids[:, None] >= start_tile[None, :]).sum(
        axis=1).astype(jnp.int32) - 1
    total_tiles = jnp.sum(padded) // TILE
    tile_live = (tile_ids < total_tiles).astype(jnp.int32)
    e_last = jnp.max(jnp.where(counts > 0, jnp.arange(E, dtype=jnp.int32), 0))
    tile_expert = jnp.where(tile_live != 0, tile_expert, e_last)

    pair_tok = (jnp.arange(n_pairs, dtype=jnp.int32) // TOP_K)
    row_ids = jnp.zeros((n_pad,), jnp.int32).at[pos].set(pair_tok)
    x_sorted = x_flat[row_ids]

    y_sorted = _grouped_ffn(x_sorted, W1, b1, W2, b2,
                            tile_expert, tile_live, n_pad)

    pos2 = pos.reshape(n_tok, TOP_K)
    out = (top_k_probs[:, 0:1] * y_sorted[pos2[:, 0]]
           + top_k_probs[:, 1:2] * y_sorted[pos2[:, 1]])
    return out.reshape(bsz, seq, d)


# speedup vs baseline: 1.0977x; 1.0977x over previous
import jax
import jax.numpy as jnp
from jax.experimental import pallas as pl
from jax.experimental.pallas import tpu as pltpu

E = 8
TOP_K = 2
TILE = 256
FF_CHUNK = 1024


def _ffn_body(te_ref, tl_ref, x_ref, W1_ref, b1_ref, W2_ref, b2_ref, o_ref,
              acc_ref):
    ff = pl.program_id(0)
    i = pl.program_id(1)
    nff = pl.num_programs(0)

    @pl.when(tl_ref[i] != 0)
    def _compute():
        h = jnp.dot(x_ref[...], W1_ref[0].astype(jnp.bfloat16),
                    preferred_element_type=jnp.float32)
        h = h + b1_ref[0, 0][None, :]
        h = 0.5 * h * (1.0 + jax.lax.erf(h * 0.7071067811865476))
        y = jnp.dot(h.astype(jnp.bfloat16), W2_ref[0].astype(jnp.bfloat16),
                    preferred_element_type=jnp.float32)
        sl = pl.ds(i * TILE, TILE)

        @pl.when(ff == 0)
        def _init():
            acc_ref[sl, :] = y + b2_ref[0, 0][None, :]

        @pl.when(jnp.logical_and(ff != 0, ff != nff - 1))
        def _acc():
            acc_ref[sl, :] += y

        @pl.when(ff == nff - 1)
        def _flush():
            o_ref[...] = acc_ref[sl, :] + y


def _grouped_ffn(x_sorted, W1, b1, W2, b2, tile_expert, tile_live, n_pad):
    d_model = x_sorted.shape[1]
    d_ff = W1.shape[2]
    nt = n_pad // TILE
    nff = d_ff // FF_CHUNK
    grid_spec = pltpu.PrefetchScalarGridSpec(
        num_scalar_prefetch=2,
        grid=(nff, nt),
        in_specs=[
            pl.BlockSpec((TILE, d_model), lambda ff, i, te, tl: (i, 0)),
            pl.BlockSpec((1, d_model, FF_CHUNK),
                         lambda ff, i, te, tl: (te[i], 0, ff)),
            pl.BlockSpec((1, 1, FF_CHUNK), lambda ff, i, te, tl: (te[i], 0, ff)),
            pl.BlockSpec((1, FF_CHUNK, d_model),
                         lambda ff, i, te, tl: (te[i], ff, 0)),
            pl.BlockSpec((1, 1, d_model), lambda ff, i, te, tl: (te[i], 0, 0)),
        ],
        out_specs=pl.BlockSpec(
            (TILE, d_model),
            lambda ff, i, te, tl: (jnp.where(ff == nff - 1, i, nt), 0)),
        scratch_shapes=[pltpu.VMEM((n_pad, d_model), jnp.float32)],
    )
    return pl.pallas_call(
        _ffn_body,
        grid_spec=grid_spec,
        out_shape=jax.ShapeDtypeStruct((n_pad + TILE, d_model), jnp.float32),
        compiler_params=pltpu.CompilerParams(
            dimension_semantics=("arbitrary", "arbitrary")),
    )(tile_expert, tile_live, x_sorted, W1,
      b1.reshape(b1.shape[0], 1, d_ff), W2,
      b2.reshape(b2.shape[0], 1, d_model))


def kernel(x, Wg, W1, b1, W2, b2):
    bsz, seq, d = x.shape
    x_flat = x.reshape(-1, d)
    n_tok = bsz * seq
    n_pairs = n_tok * TOP_K
    nt = n_pairs // TILE + E
    n_pad = nt * TILE

    gate_scores = x_flat @ Wg
    gate_probs = jax.nn.softmax(gate_scores, axis=-1)
    top_k_probs, top_k_idx = jax.lax.top_k(gate_probs, TOP_K)
    top_k_probs = top_k_probs / jnp.sum(top_k_probs, axis=-1, keepdims=True)

    e_flat = top_k_idx.reshape(-1).astype(jnp.int32)
    p_flat = top_k_probs.reshape(-1)
    onehot = (e_flat[:, None] == jnp.arange(E, dtype=jnp.int32)[None, :])
    onehot = onehot.astype(jnp.int32)
    rank = (jnp.cumsum(onehot, axis=0) - onehot)
    rank = (rank * onehot).sum(axis=1)
    counts = onehot.sum(axis=0)
    padded = ((counts + TILE - 1) // TILE) * TILE
    starts = jnp.concatenate(
        [jnp.zeros((1,), jnp.int32), jnp.cumsum(padded).astype(jnp.int32)])[:E]
    pos = starts[e_flat] + rank

    start_tile = starts // TILE
    tile_ids = jnp.arange(nt, dtype=jnp.int32)
    tile_expert = (tile_ids[:, None] >= start_tile[None, :]).sum(
        axis=1).astype(jnp.int32) - 1
    total_tiles = jnp.sum(padded) // TILE
    tile_live = (tile_ids < total_tiles).astype(jnp.int32)
    e_last = jnp.max(jnp.where(counts > 0, jnp.arange(E, dtype=jnp.int32), 0))
    tile_expert = jnp.where(tile_live != 0, tile_expert, e_last)

    pair_tok = (jnp.arange(n_pairs, dtype=jnp.int32) // TOP_K)
    row_ids = jnp.zeros((n_pad,), jnp.int32).at[pos].set(pair_tok)
    x_sorted = x_flat.astype(jnp.bfloat16)[row_ids]

    y_sorted = _grouped_ffn(x_sorted, W1, b1, W2, b2,
                            tile_expert, tile_live, n_pad)

    pos2 = pos.reshape(n_tok, TOP_K)
    out = (top_k_probs[:, 0:1] * y_sorted[pos2[:, 0]]
           + top_k_probs[:, 1:2] * y_sorted[pos2[:, 1]])
    return out.reshape(bsz, seq, d)


# speedup vs baseline: 1.1139x; 1.0147x over previous
import jax
import jax.numpy as jnp
from jax.experimental import pallas as pl
from jax.experimental.pallas import tpu as pltpu

E = 8
TOP_K = 2
TILE = 256
FF_CHUNK = 1024


def _ffn_body(te_ref, tl_ref, x_ref, W1_ref, b1_ref, W2_ref, b2_ref, o_ref,
              acc_ref):
    ff = pl.program_id(0)
    i = pl.program_id(1)
    nff = pl.num_programs(0)

    @pl.when(tl_ref[i] != 0)
    def _compute():
        h = jnp.dot(x_ref[...], W1_ref[0].astype(jnp.bfloat16),
                    preferred_element_type=jnp.float32)
        h = h + b1_ref[0, 0][None, :]
        h = 0.5 * h * (1.0 + jax.lax.erf(h * 0.7071067811865476))
        y = jnp.dot(h.astype(jnp.bfloat16), W2_ref[0].astype(jnp.bfloat16),
                    preferred_element_type=jnp.float32)
        sl = pl.ds(i * TILE, TILE)

        @pl.when(ff == 0)
        def _init():
            acc_ref[sl, :] = y + b2_ref[0, 0][None, :]

        @pl.when(jnp.logical_and(ff != 0, ff != nff - 1))
        def _acc():
            acc_ref[sl, :] += y

        @pl.when(ff == nff - 1)
        def _flush():
            o_ref[...] = acc_ref[sl, :] + y


def _grouped_ffn(x_sorted, W1, b1, W2, b2, tile_expert, tile_live, n_pad):
    d_model = x_sorted.shape[1]
    d_ff = W1.shape[2]
    nt = n_pad // TILE
    nff = d_ff // FF_CHUNK
    grid_spec = pltpu.PrefetchScalarGridSpec(
        num_scalar_prefetch=2,
        grid=(nff, nt),
        in_specs=[
            pl.BlockSpec((TILE, d_model), lambda ff, i, te, tl: (i, 0)),
            pl.BlockSpec((1, d_model, FF_CHUNK),
                         lambda ff, i, te, tl: (te[i], 0, ff)),
            pl.BlockSpec((1, 1, FF_CHUNK), lambda ff, i, te, tl: (te[i], 0, ff)),
            pl.BlockSpec((1, FF_CHUNK, d_model),
                         lambda ff, i, te, tl: (te[i], ff, 0)),
            pl.BlockSpec((1, 1, d_model), lambda ff, i, te, tl: (te[i], 0, 0)),
        ],
        out_specs=pl.BlockSpec(
            (TILE, d_model),
            lambda ff, i, te, tl: (jnp.where(ff == nff - 1, i, nt), 0)),
        scratch_shapes=[pltpu.VMEM((n_pad, d_model), jnp.float32)],
    )
    return pl.pallas_call(
        _ffn_body,
        grid_spec=grid_spec,
        out_shape=jax.ShapeDtypeStruct((n_pad + TILE, d_model), jnp.float32),
        compiler_params=pltpu.CompilerParams(
            dimension_semantics=("arbitrary", "arbitrary")),
    )(tile_expert, tile_live, x_sorted, W1,
      b1.reshape(b1.shape[0], 1, d_ff), W2,
      b2.reshape(b2.shape[0], 1, d_model))


def _router_body(s_ref, pos_ref, tkp_ref, te_ref, tl_ref):
    n_tok, n_exp = s_ref.shape
    n_pairs = n_tok * TOP_K
    nt = n_pairs // TILE + n_exp
    s = s_ref[...]
    lane = jax.lax.broadcasted_iota(jnp.int32, (n_tok, n_exp), 1)
    m1 = jnp.max(s, axis=1, keepdims=True)
    i1 = jnp.min(jnp.where(s == m1, lane, n_exp), axis=1, keepdims=True)
    s2 = jnp.where(lane == i1, -3.4e38, s)
    m2 = jnp.max(s2, axis=1, keepdims=True)
    i2 = jnp.min(jnp.where(s2 == m2, lane, n_exp), axis=1, keepdims=True)
    t = jnp.exp(m2 - m1)
    denom = 1.0 + t
    tkp_ref[...] = jnp.concatenate([1.0 / denom, t / denom], axis=1)

    ef = jnp.concatenate([i1, i2], axis=0)
    lane8 = jax.lax.broadcasted_iota(jnp.int32, (n_pairs, n_exp), 1)
    oh = (ef == lane8).astype(jnp.float32)
    blk = 512
    tri = (jax.lax.broadcasted_iota(jnp.int32, (blk, blk), 0)
           > jax.lax.broadcasted_iota(jnp.int32, (blk, blk), 1)
           ).astype(jnp.float32)
    carry = jnp.zeros((1, n_exp), jnp.float32)
    ranks = []
    for b in range(n_pairs // blk):
        ohb = oh[b * blk:(b + 1) * blk, :]
        rb = jnp.dot(tri, ohb, preferred_element_type=jnp.float32) + carry
        ranks.append(jnp.sum(rb * ohb, axis=1, keepdims=True))
        carry = carry + jnp.sum(ohb, axis=0, keepdims=True)
    rank = jnp.concatenate(ranks, axis=0)
    counts = carry
    padded = jnp.ceil(counts * (1.0 / TILE)) * TILE
    triu = (jax.lax.broadcasted_iota(jnp.int32, (n_exp, n_exp), 0)
            < jax.lax.broadcasted_iota(jnp.int32, (n_exp, n_exp), 1)
            ).astype(jnp.float32)
    starts = jnp.dot(padded, triu, preferred_element_type=jnp.float32)
    pos_f = jnp.sum(oh * starts, axis=1, keepdims=True) + rank
    pos_ref[...] = pos_f.astype(jnp.int32)

    start_tile = starts * (1.0 / TILE)
    total_tiles = jnp.sum(padded) * (1.0 / TILE)
    tgrid = jax.lax.broadcasted_iota(jnp.int32, (nt, n_exp), 0).astype(
        jnp.float32)
    te = jnp.sum((tgrid >= start_tile).astype(jnp.float32), axis=1,
                 keepdims=True) - 1.0
    tl = (jax.lax.broadcasted_iota(jnp.int32, (nt, 1), 0).astype(jnp.float32)
          < total_tiles)
    lane_f = jax.lax.broadcasted_iota(jnp.int32, (1, n_exp), 1).astype(
        jnp.float32)
    e_last = jnp.max(jnp.where(counts > 0, lane_f, 0.0))
    te = jnp.where(tl, te, e_last)
    te_ref[...] = te.astype(jnp.int32)
    tl_ref[...] = tl.astype(jnp.int32)


def _router(scores, n_pairs, nt):
    n_tok, n_exp = scores.shape
    return pl.pallas_call(
        _router_body,
        out_shape=[
            jax.ShapeDtypeStruct((n_pairs, 1), jnp.int32),
            jax.ShapeDtypeStruct((n_tok, TOP_K), jnp.float32),
            jax.ShapeDtypeStruct((nt, 1), jnp.int32),
            jax.ShapeDtypeStruct((nt, 1), jnp.int32),
        ],
    )(scores)


def kernel(x, Wg, W1, b1, W2, b2):
    bsz, seq, d = x.shape
    x_flat = x.reshape(-1, d)
    n_tok = bsz * seq
    n_pairs = n_tok * TOP_K
    nt = n_pairs // TILE + E
    n_pad = nt * TILE

    gate_scores = x_flat @ Wg
    pos2d, top_k_probs, te2d, tl2d = _router(gate_scores, n_pairs, nt)
    pos = pos2d.reshape(n_pairs)
    tile_expert = te2d.reshape(nt)
    tile_live = tl2d.reshape(nt)

    tok_ids = jnp.arange(n_tok, dtype=jnp.int32)
    pair_tok = jnp.concatenate([tok_ids, tok_ids])
    row_ids = jnp.zeros((n_pad,), jnp.int32).at[pos].set(pair_tok)
    x_sorted = x_flat.astype(jnp.bfloat16)[row_ids]

    y_sorted = _grouped_ffn(x_sorted, W1, b1, W2, b2,
                            tile_expert, tile_live, n_pad)

    out = (top_k_probs[:, 0:1] * y_sorted[pos[:n_tok]]
           + top_k_probs[:, 1:2] * y_sorted[pos[n_tok:]])
    return out.reshape(bsz, seq, d)


# speedup vs baseline: 1.2137x; 1.0897x over previous
import functools

import jax
import jax.numpy as jnp
from jax.experimental import pallas as pl
from jax.experimental.pallas import tpu as pltpu
from jax.experimental.pallas import tpu_sc as plsc

E = 8
TOP_K = 2
TILE = 256
FF_CHUNK = 1024


def _ffn_body(te_ref, tl_ref, x_ref, W1_ref, b1_ref, W2_ref, b2_ref, o_ref,
              acc_ref):
    ff = pl.program_id(0)
    i = pl.program_id(1)
    nff = pl.num_programs(0)

    @pl.when(tl_ref[i] != 0)
    def _compute():
        h = jnp.dot(x_ref[...].astype(jnp.bfloat16),
                    W1_ref[0].astype(jnp.bfloat16),
                    preferred_element_type=jnp.float32)
        h = h + b1_ref[0, 0][None, :]
        h = 0.5 * h * (1.0 + jax.lax.erf(h * 0.7071067811865476))
        y = jnp.dot(h.astype(jnp.bfloat16), W2_ref[0].astype(jnp.bfloat16),
                    preferred_element_type=jnp.float32)
        sl = pl.ds(i * TILE, TILE)

        @pl.when(ff == 0)
        def _init():
            acc_ref[sl, :] = y + b2_ref[0, 0][None, :]

        @pl.when(jnp.logical_and(ff != 0, ff != nff - 1))
        def _acc():
            acc_ref[sl, :] += y

        @pl.when(ff == nff - 1)
        def _flush():
            o_ref[...] = acc_ref[sl, :] + y


def _grouped_ffn(x_sorted, W1, b1, W2, b2, tile_expert, tile_live, n_pad):
    d_model = x_sorted.shape[1]
    d_ff = W1.shape[2]
    nt = n_pad // TILE
    nff = d_ff // FF_CHUNK
    grid_spec = pltpu.PrefetchScalarGridSpec(
        num_scalar_prefetch=2,
        grid=(nff, nt),
        in_specs=[
            pl.BlockSpec((TILE, d_model), lambda ff, i, te, tl: (i, 0)),
            pl.BlockSpec((1, d_model, FF_CHUNK),
                         lambda ff, i, te, tl: (te[i], 0, ff)),
            pl.BlockSpec((1, 1, FF_CHUNK), lambda ff, i, te, tl: (te[i], 0, ff)),
            pl.BlockSpec((1, FF_CHUNK, d_model),
                         lambda ff, i, te, tl: (te[i], ff, 0)),
            pl.BlockSpec((1, 1, d_model), lambda ff, i, te, tl: (te[i], 0, 0)),
        ],
        out_specs=pl.BlockSpec(
            (TILE, d_model),
            lambda ff, i, te, tl: (jnp.where(ff == nff - 1, i, nt), 0)),
        scratch_shapes=[pltpu.VMEM((n_pad, d_model), jnp.float32)],
    )
    return pl.pallas_call(
        _ffn_body,
        grid_spec=grid_spec,
        out_shape=jax.ShapeDtypeStruct((n_pad + TILE, d_model), jnp.float32),
        compiler_params=pltpu.CompilerParams(
            dimension_semantics=("arbitrary", "arbitrary")),
    )(tile_expert, tile_live, x_sorted, W1,
      b1.reshape(b1.shape[0], 1, d_ff), W2,
      b2.reshape(b2.shape[0], 1, d_model))


def _router_body(s_ref, pos_ref, tkp_ref, te_ref, tl_ref):
    n_tok, n_exp = s_ref.shape
    n_pairs = n_tok * TOP_K
    nt = n_pairs // TILE + n_exp
    s = s_ref[...]
    lane = jax.lax.broadcasted_iota(jnp.int32, (n_tok, n_exp), 1)
    m1 = jnp.max(s, axis=1, keepdims=True)
    i1 = jnp.min(jnp.where(s == m1, lane, n_exp), axis=1, keepdims=True)
    s2 = jnp.where(lane == i1, -3.4e38, s)
    m2 = jnp.max(s2, axis=1, keepdims=True)
    i2 = jnp.min(jnp.where(s2 == m2, lane, n_exp), axis=1, keepdims=True)
    t = jnp.exp(m2 - m1)
    denom = 1.0 + t
    tkp_ref[...] = jnp.concatenate([1.0 / denom, t / denom], axis=1)

    ef = jnp.concatenate([i1, i2], axis=0)
    lane8 = jax.lax.broadcasted_iota(jnp.int32, (n_pairs, n_exp), 1)
    oh = (ef == lane8).astype(jnp.float32)
    blk = 512
    tri = (jax.lax.broadcasted_iota(jnp.int32, (blk, blk), 0)
           > jax.lax.broadcasted_iota(jnp.int32, (blk, blk), 1)
           ).astype(jnp.float32)
    carry = jnp.zeros((1, n_exp), jnp.float32)
    ranks = []
    for b in range(n_pairs // blk):
        ohb = oh[b * blk:(b + 1) * blk, :]
        rb = jnp.dot(tri, ohb, preferred_element_type=jnp.float32) + carry
        ranks.append(jnp.sum(rb * ohb, axis=1, keepdims=True))
        carry = carry + jnp.sum(ohb, axis=0, keepdims=True)
    rank = jnp.concatenate(ranks, axis=0)
    counts = carry
    padded = jnp.ceil(counts * (1.0 / TILE)) * TILE
    triu = (jax.lax.broadcasted_iota(jnp.int32, (n_exp, n_exp), 0)
            < jax.lax.broadcasted_iota(jnp.int32, (n_exp, n_exp), 1)
            ).astype(jnp.float32)
    starts = jnp.dot(padded, triu, preferred_element_type=jnp.float32)
    pos_f = jnp.sum(oh * starts, axis=1, keepdims=True) + rank
    pos_ref[...] = pos_f.astype(jnp.int32)

    start_tile = starts * (1.0 / TILE)
    total_tiles = jnp.sum(padded) * (1.0 / TILE)
    tgrid = jax.lax.broadcasted_iota(jnp.int32, (nt, n_exp), 0).astype(
        jnp.float32)
    te = jnp.sum((tgrid >= start_tile).astype(jnp.float32), axis=1,
                 keepdims=True) - 1.0
    tl = (jax.lax.broadcasted_iota(jnp.int32, (nt, 1), 0).astype(jnp.float32)
          < total_tiles)
    lane_f = jax.lax.broadcasted_iota(jnp.int32, (1, n_exp), 1).astype(
        jnp.float32)
    e_last = jnp.max(jnp.where(counts > 0, lane_f, 0.0))
    te = jnp.where(tl, te, e_last)
    te_ref[...] = te.astype(jnp.int32)
    tl_ref[...] = tl.astype(jnp.int32)


def _dispatch_sc(x_flat, pos, pair_tok, n_pad):
    n_tok, d = x_flat.shape
    n_pairs = pos.shape[0]
    info = plsc.get_sparse_core_info()
    nw = info.num_cores * info.num_subcores
    per_w = n_pairs // nw
    ch = 32
    nch = per_w // ch
    mesh = plsc.VectorSubcoreMesh(core_axis_name="c", subcore_axis_name="s")

    @functools.partial(
        pl.kernel, mesh=mesh,
        out_type=jax.ShapeDtypeStruct((n_pad, d), jnp.float32),
        scratch_types=[
            pltpu.VMEM((ch,), jnp.int32),
            pltpu.VMEM((ch,), jnp.int32),
            pltpu.VMEM((ch, d), jnp.float32),
            pltpu.SemaphoreType.DMA,
        ],
    )
    def k(x_hbm, pos_hbm, tok_hbm, out_hbm, tok_c, pos_c, rows_v, sem):
        wid = (jax.lax.axis_index("s") * info.num_cores
               + jax.lax.axis_index("c"))
        base = wid * per_w

        def body(c, carry):
            off = base + c * ch
            pltpu.sync_copy(tok_hbm.at[pl.ds(off, ch)], tok_c)
            pltpu.async_copy(x_hbm.at[tok_c], rows_v, sem).wait()
            pltpu.sync_copy(pos_hbm.at[pl.ds(off, ch)], pos_c)
            pltpu.async_copy(rows_v, out_hbm.at[pos_c], sem).wait()
            return carry

        jax.lax.fori_loop(0, nch, body, 0)

    return k(x_flat, pos, pair_tok)


def _router(scores, n_pairs, nt):
    n_tok, n_exp = scores.shape
    return pl.pallas_call(
        _router_body,
        out_shape=[
            jax.ShapeDtypeStruct((n_pairs, 1), jnp.int32),
            jax.ShapeDtypeStruct((n_tok, TOP_K), jnp.float32),
            jax.ShapeDtypeStruct((nt, 1), jnp.int32),
            jax.ShapeDtypeStruct((nt, 1), jnp.int32),
        ],
    )(scores)


def kernel(x, Wg, W1, b1, W2, b2):
    bsz, seq, d = x.shape
    x_flat = x.reshape(-1, d)
    n_tok = bsz * seq
    n_pairs = n_tok * TOP_K
    nt = n_pairs // TILE + E
    n_pad = nt * TILE

    gate_scores = x_flat @ Wg
    pos2d, top_k_probs, te2d, tl2d = _router(gate_scores, n_pairs, nt)
    pos = pos2d.reshape(n_pairs)
    tile_expert = te2d.reshape(nt)
    tile_live = tl2d.reshape(nt)

    tok_ids = jnp.arange(n_tok, dtype=jnp.int32)
    pair_tok = jnp.concatenate([tok_ids, tok_ids])
    x_sorted = _dispatch_sc(x_flat, pos, pair_tok, n_pad)

    y_sorted = _grouped_ffn(x_sorted, W1, b1, W2, b2,
                            tile_expert, tile_live, n_pad)

    out = (top_k_probs[:, 0:1] * y_sorted[pos[:n_tok]]
           + top_k_probs[:, 1:2] * y_sorted[pos[n_tok:]])
    return out.reshape(bsz, seq, d)
